# baseline (speedup 1.0000x reference)
import functools

import jax
import jax.numpy as jnp
from jax import lax
from jax.experimental import pallas as pl
from jax.experimental.pallas import tpu as pltpu
from jax.experimental.pallas import tpu_sc as plsc

NUM_SAMPLED = 255
S_PAD = 256
LANES = 128

_CHUNK = 128


@functools.lru_cache(maxsize=None)
def _make_sc_gather(V4, B):
    info = plsc.get_sparse_core_info()
    NC, NS = info.num_cores, info.num_subcores
    NW = NC * NS
    assert B % _CHUNK == 0
    n_chunks = B // _CHUNK
    n_extra = n_chunks - NW
    assert 0 <= n_extra <= NW

    mesh = plsc.VectorSubcoreMesh(core_axis_name="c", subcore_axis_name="s")

    @functools.partial(
        pl.kernel,
        mesh=mesh,
        out_type=jax.ShapeDtypeStruct((B, LANES), jnp.float32),
        scratch_types=[
            pltpu.VMEM((_CHUNK,), jnp.int32),
            pltpu.VMEM((_CHUNK, LANES), jnp.float32),
            pltpu.SemaphoreType.DMA,
        ],
    )
    def gather(table_hbm, idx_hbm, out_hbm, idx_v, rows_v, sem):
        wid = lax.axis_index("s") * NC + lax.axis_index("c")

        def do_chunk(chunk_id):
            base = pl.multiple_of(chunk_id * _CHUNK, _CHUNK)
            pltpu.sync_copy(idx_hbm.at[pl.ds(base, _CHUNK)], idx_v)
            pltpu.async_copy(table_hbm.at[idx_v], rows_v, sem).wait()
            pltpu.sync_copy(rows_v, out_hbm.at[pl.ds(base, _CHUNK)])

        do_chunk(wid)
        if n_extra:
            @pl.when(wid < n_extra)
            def _():
                do_chunk(NW + wid)

    return gather


def _select_group(rows, rem):
    out = None
    for g in range(4):
        mask = (rem == g).astype(jnp.float32)
        part = rows[:, g * 32:(g + 1) * 32] * mask
        out = part if out is None else out + part
    return out


def _dense_body(logv1_ref, idx_ref, uv_ref, gt_ref, sw_ref, rems_ref,
                cmp_ref, nlse_ref, out_ref):
    logv1 = logv1_ref[0]
    idxc = idx_ref[...]
    c = idxc.astype(jnp.float32)
    p_true = (jnp.log(c + 2.0) - jnp.log(c + 1.0)) / logv1
    q = NUM_SAMPLED * jnp.log1p(-p_true)
    lte_small = jnp.log(-q) + jnp.log1p(
        q * (0.5 + q * (1.0 / 6.0 + q * (1.0 / 24.0))))
    lte_big = jnp.log(1.0 - jnp.exp(q))
    lte = jnp.where(q > -0.1, lte_small, lte_big)

    uv = uv_ref[...]
    tw = _select_group(gt_ref[...], idxc % 4)
    tl = jnp.sum(uv * tw, axis=1, keepdims=True) - lte

    swsel = _select_group(sw_ref[...], rems_ref[...])
    slt = lax.dot_general(
        uv, swsel, (((1,), (1,)), ((), ())),
        preferred_element_type=jnp.float32,
    )
    slt = slt + nlse_ref[...]
    acc = (cmp_ref[...] == idxc).astype(jnp.float32)
    slt = slt - acc * 1e9

    m = jnp.maximum(jnp.max(slt, axis=1, keepdims=True), tl)
    z = jnp.sum(jnp.exp(slt - m), axis=1, keepdims=True) + jnp.exp(tl - m)
    out_ref[...] = jnp.log(z) + m - tl


def _dense(logv1, idxc, uv, gathered, rems, cmp, nlse, *, interpret=False):
    B, D = uv.shape
    BB = 512
    grid = (B // BB,)
    sw_block = B // S_PAD
    return pl.pallas_call(
        _dense_body,
        grid=grid,
        in_specs=[
            pl.BlockSpec(memory_space=pltpu.SMEM),
            pl.BlockSpec((BB, 1), lambda i: (i, 0)),
            pl.BlockSpec((BB, D), lambda i: (i, 0)),
            pl.BlockSpec((BB, LANES), lambda i: (i, 0)),
            pl.BlockSpec((S_PAD, LANES), lambda i: (sw_block, 0)),
            pl.BlockSpec((S_PAD, 1), lambda i: (0, 0)),
            pl.BlockSpec((1, S_PAD), lambda i: (0, 0)),
            pl.BlockSpec((1, S_PAD), lambda i: (0, 0)),
        ],
        out_specs=pl.BlockSpec((BB, 1), lambda i: (i, 0)),
        out_shape=jax.ShapeDtypeStruct((B, 1), jnp.float32),
        interpret=interpret,
    )(logv1, idxc, uv, gathered, gathered, rems, cmp, nlse)


def kernel(item_embeddings, user_vec, item_idx, zero_bias):
    V, D = item_embeddings.shape
    B = user_vec.shape[0]
    BG = B + S_PAD
    idx = item_idx.reshape(-1).astype(jnp.int32)

    logv1 = jnp.log(jnp.float32(V) + 1.0)
    skey = jax.random.fold_in(jax.random.key(0), 12345)
    u = jax.random.uniform(skey, (NUM_SAMPLED,), dtype=jnp.float32)
    s = jnp.floor(jnp.exp(u * logv1)) - 1.0
    sampled = jnp.clip(s, 0, V - 1).astype(jnp.int32)
    cs = sampled.astype(jnp.float32)
    p_samp = (jnp.log(cs + 2.0) - jnp.log(cs + 1.0)) / logv1
    nlse = -jnp.log(-jnp.expm1(NUM_SAMPLED * jnp.log1p(-p_samp)))
    nlse_pad = jnp.concatenate([nlse, jnp.full((1,), -1e30, jnp.float32)])
    cmp_pad = jnp.concatenate([sampled, jnp.full((1,), -1, jnp.int32)])
    rems_pad = jnp.concatenate([sampled % 4, jnp.zeros((1,), jnp.int32)])
    fill = jnp.zeros((BG - B - NUM_SAMPLED,), jnp.int32)
    all_idx = jnp.concatenate([idx, sampled, fill])

    assert (V * D) % LANES == 0
    table4 = item_embeddings.reshape((V * D) // LANES, LANES)
    gathered = _make_sc_gather(table4.shape[0], BG)(
        table4, all_idx // (LANES // D))

    return _dense(
        logv1.reshape(1),
        idx.reshape(B, 1),
        user_vec,
        gathered,
        rems_pad.reshape(S_PAD, 1),
        cmp_pad.reshape(1, S_PAD),
        nlse_pad.reshape(1, S_PAD),
    )
